# baseline (speedup 1.0000x reference)
import functools

import jax
import jax.numpy as jnp
from jax.experimental import pallas as pl


def _dcn_loss_kernel(e_ref, ct_ref, ch_ref, out_ref, *, n_total):
    i = pl.program_id(0)

    e = e_ref[...]
    ct = ct_ref[...]
    ch = ch_ref[...]

    eb = e.astype(jnp.bfloat16)
    k = ct.shape[1]
    kc = k // 4
    rowmax = None
    for j in range(4):
        prod = jax.lax.dot_general(
            eb, ct[:, j * kc:(j + 1) * kc], (((1,), (0,)), ((), ())),
            preferred_element_type=jnp.float32,
        )
        m = jnp.max(prod - ch[:, j * kc:(j + 1) * kc], axis=1, keepdims=True)
        rowmax = m if rowmax is None else jnp.maximum(rowmax, m)
    partial = (jnp.sum(e * e) - 2.0 * jnp.sum(rowmax)) / n_total

    @pl.when(i == 0)
    def _():
        out_ref[...] = jnp.zeros((1, 1), jnp.float32)

    out_ref[...] += partial.reshape(1, 1)


def kernel(embedded, centers):
    n, d = embedded.shape
    k, _ = centers.shape
    bn = 1024
    num_blocks = n // bn

    ct = centers.T.astype(jnp.bfloat16)
    ch = 0.5 * jnp.sum(centers * centers, axis=1)[None, :]

    out = pl.pallas_call(
        functools.partial(_dcn_loss_kernel, n_total=float(n)),
        grid=(num_blocks,),
        in_specs=[
            pl.BlockSpec((bn, d), lambda i: (i, 0)),
            pl.BlockSpec((d, k), lambda i: (0, 0)),
            pl.BlockSpec((1, k), lambda i: (0, 0)),
        ],
        out_specs=pl.BlockSpec((1, 1), lambda i: (0, 0)),
        out_shape=jax.ShapeDtypeStruct((1, 1), jnp.float32),
    )(embedded, ct, ch)
    return out[0, 0]


# speedup vs baseline: 4.8413x; 4.8413x over previous
import functools

import jax
import jax.numpy as jnp
from jax.experimental import pallas as pl
from jax.experimental.pallas import tpu as pltpu


def _dcn_loss_kernel(e_ref, ct_ref, ch_ref, out_ref, m_ref, *, n_total,
                     num_blocks):
    i = pl.program_id(0)
    parity = jax.lax.rem(i, 2)

    e = e_ref[...]
    ct = ct_ref[...]
    ch = ch_ref[...]

    q = m_ref[1 - parity, :, :]
    rowmax = jnp.max(q, axis=1, keepdims=True)
    prev = jnp.where(i > 0, jnp.sum(rowmax) * -2.0, 0.0)

    eb = e.astype(jnp.bfloat16)
    k = ct.shape[1]
    kc = k // 4
    m = None
    for j in range(4):
        prod = jax.lax.dot_general(
            eb, ct[:, j * kc:(j + 1) * kc], (((1,), (0,)), ((), ())),
            preferred_element_type=jnp.float32,
        )
        t = prod - ch[:, j * kc:(j + 1) * kc]
        pm = jnp.maximum(t[:, :128], t[:, 128:])
        m = pm if m is None else jnp.maximum(m, pm)
    m_ref[parity, :, :] = m

    e2 = jnp.where(i < num_blocks, jnp.sum(e * e), 0.0)

    @pl.when(i == 0)
    def _():
        out_ref[...] = jnp.zeros((1, 1), jnp.float32)

    out_ref[...] += ((e2 + prev) / n_total).reshape(1, 1)


def kernel(embedded, centers):
    n, d = embedded.shape
    k, _ = centers.shape
    bn = 1024
    num_blocks = n // bn

    ct = centers.T.astype(jnp.bfloat16)
    ch = 0.5 * jnp.sum(centers * centers, axis=1)[None, :]

    out = pl.pallas_call(
        functools.partial(_dcn_loss_kernel, n_total=float(n),
                          num_blocks=num_blocks),
        grid=(num_blocks + 1,),
        in_specs=[
            pl.BlockSpec((bn, d),
                         lambda i: (jnp.minimum(i, num_blocks - 1), 0)),
            pl.BlockSpec((d, k), lambda i: (0, 0)),
            pl.BlockSpec((1, k), lambda i: (0, 0)),
        ],
        out_specs=pl.BlockSpec((1, 1), lambda i: (0, 0)),
        out_shape=jax.ShapeDtypeStruct((1, 1), jnp.float32),
        scratch_shapes=[pltpu.VMEM((2, bn, 128), jnp.float32)],
    )(embedded, ct, ch)
    return out[0, 0]


# speedup vs baseline: 4.8508x; 1.0020x over previous
import functools

import jax
import jax.numpy as jnp
from jax.experimental import pallas as pl
from jax.experimental.pallas import tpu as pltpu


def _dcn_loss_kernel(e_ref, c_ref, out_ref, ct_ref, ch_ref, m_ref, *,
                     n_total, num_blocks):
    i = pl.program_id(0)
    parity = jax.lax.rem(i, 2)

    @pl.when(i == 0)
    def _():
        c = c_ref[...]
        ctf = c.T
        ct_ref[...] = ctf.astype(jnp.bfloat16)
        ch_ref[...] = 0.5 * jnp.sum(ctf * ctf, axis=0, keepdims=True)
        out_ref[...] = jnp.zeros((1, 1), jnp.float32)

    e = e_ref[...]
    ct = ct_ref[...]
    ch = ch_ref[...]

    q = m_ref[1 - parity, :, :]
    rowmax = jnp.max(q, axis=1, keepdims=True)
    prev = jnp.where(i > 0, jnp.sum(rowmax) * -2.0, 0.0)

    eb = e.astype(jnp.bfloat16)
    k = ct.shape[1]
    kc = k // 4
    m = None
    for j in range(4):
        prod = jax.lax.dot_general(
            eb, ct[:, j * kc:(j + 1) * kc], (((1,), (0,)), ((), ())),
            preferred_element_type=jnp.float32,
        )
        t = prod - ch[:, j * kc:(j + 1) * kc]
        pm = jnp.maximum(t[:, :128], t[:, 128:])
        m = pm if m is None else jnp.maximum(m, pm)
    m_ref[parity, :, :] = m

    e2 = jnp.where(i < num_blocks, jnp.sum(e * e), 0.0)

    out_ref[...] += ((e2 + prev) / n_total).reshape(1, 1)


def kernel(embedded, centers):
    n, d = embedded.shape
    k, _ = centers.shape
    bn = 1024
    num_blocks = n // bn

    out = pl.pallas_call(
        functools.partial(_dcn_loss_kernel, n_total=float(n),
                          num_blocks=num_blocks),
        grid=(num_blocks + 1,),
        in_specs=[
            pl.BlockSpec((bn, d),
                         lambda i: (jnp.minimum(i, num_blocks - 1), 0)),
            pl.BlockSpec((k, d), lambda i: (0, 0)),
        ],
        out_specs=pl.BlockSpec((1, 1), lambda i: (0, 0)),
        out_shape=jax.ShapeDtypeStruct((1, 1), jnp.float32),
        scratch_shapes=[
            pltpu.VMEM((d, k), jnp.bfloat16),
            pltpu.VMEM((1, k), jnp.float32),
            pltpu.VMEM((2, bn, 128), jnp.float32),
        ],
    )(embedded, centers)
    return out[0, 0]


# speedup vs baseline: 5.3356x; 1.0999x over previous
import functools

import jax
import jax.numpy as jnp
from jax.experimental import pallas as pl
from jax.experimental.pallas import tpu as pltpu


def _dcn_loss_kernel(e_ref, c_ref, out_ref, ct_ref, ch_ref, m_ref, *,
                     n_total, num_blocks):
    i = pl.program_id(0)
    parity = jax.lax.rem(i, 2)

    @pl.when(i == 0)
    def _():
        c = c_ref[...]
        ctf = c.T
        ct_ref[...] = ctf.astype(jnp.bfloat16)
        ch_ref[...] = 0.5 * jnp.sum(ctf * ctf, axis=0, keepdims=True)
        out_ref[...] = jnp.zeros((1, 1), jnp.float32)

    e = e_ref[...]
    ct = ct_ref[...]
    ch = ch_ref[...]

    q = m_ref[1 - parity, :, :]
    rowmax = jnp.max(q, axis=1, keepdims=True)
    prev = jnp.where(i > 0, jnp.sum(rowmax) * -2.0, 0.0)

    eb = e.astype(jnp.bfloat16)
    k = ct.shape[1]
    kc = k // 4
    m = None
    for j in range(4):
        prod = jax.lax.dot_general(
            eb, ct[:, j * kc:(j + 1) * kc], (((1,), (0,)), ((), ())),
            preferred_element_type=jnp.float32,
        )
        t = prod - ch[:, j * kc:(j + 1) * kc]
        pm = jnp.maximum(t[:, :128], t[:, 128:])
        m = pm if m is None else jnp.maximum(m, pm)
    m_ref[parity, :, :] = m

    e2 = jnp.where(i < num_blocks, jnp.sum(e * e), 0.0)

    out_ref[...] += ((e2 + prev) / n_total).reshape(1, 1)


def kernel(embedded, centers):
    n, d = embedded.shape
    k, _ = centers.shape
    bn = 2048
    num_blocks = n // bn

    out = pl.pallas_call(
        functools.partial(_dcn_loss_kernel, n_total=float(n),
                          num_blocks=num_blocks),
        grid=(num_blocks + 1,),
        in_specs=[
            pl.BlockSpec((bn, d),
                         lambda i: (jnp.minimum(i, num_blocks - 1), 0)),
            pl.BlockSpec((k, d), lambda i: (0, 0)),
        ],
        out_specs=pl.BlockSpec((1, 1), lambda i: (0, 0)),
        out_shape=jax.ShapeDtypeStruct((1, 1), jnp.float32),
        scratch_shapes=[
            pltpu.VMEM((d, k), jnp.bfloat16),
            pltpu.VMEM((1, k), jnp.float32),
            pltpu.VMEM((2, bn, 128), jnp.float32),
        ],
    )(embedded, centers)
    return out[0, 0]


# speedup vs baseline: 5.8221x; 1.0912x over previous
import functools

import jax
import jax.numpy as jnp
from jax.experimental import pallas as pl
from jax.experimental.pallas import tpu as pltpu


def _dcn_loss_kernel(e_ref, c_ref, out_ref, ct_ref, m_ref, acc_ref, *,
                     n_total, num_blocks):
    i = pl.program_id(0)
    parity = jax.lax.rem(i, 2)
    bn = e_ref.shape[0]
    k = c_ref.shape[0]

    @pl.when(i == 0)
    def _():
        c = c_ref[...]
        ctf = c.T
        ch = 0.5 * jnp.sum(ctf * ctf, axis=0, keepdims=True)
        ct_ref[...] = jnp.concatenate(
            [ctf, -ch, jnp.zeros((7, k), jnp.float32)], axis=0
        ).astype(jnp.bfloat16)
        acc_ref[0, 0] = 0.0

    e = e_ref[...]

    q = m_ref[1 - parity, :, :]
    rowmax = jnp.max(q, axis=1, keepdims=True)
    prev = jnp.where(i > 0, jnp.sum(rowmax) * -2.0, 0.0)

    eb = e.astype(jnp.bfloat16)
    ea = jnp.concatenate(
        [eb, jnp.ones((bn, 1), jnp.bfloat16),
         jnp.zeros((bn, 7), jnp.bfloat16)], axis=1)

    kc = k // 8
    m = None
    for j in range(8):
        prod = jax.lax.dot_general(
            ea, ct_ref[:, j * kc:(j + 1) * kc], (((1,), (0,)), ((), ())),
            preferred_element_type=jnp.float32,
        )
        m = prod if m is None else jnp.maximum(m, prod)
    m_ref[parity, :, :] = m

    e2 = jnp.where(i < num_blocks, jnp.sum(e * e), 0.0)

    acc_ref[0, 0] += (e2 + prev) / n_total

    @pl.when(i == num_blocks)
    def _():
        out_ref[...] = acc_ref[0, 0].reshape(1, 1)


def kernel(embedded, centers):
    n, d = embedded.shape
    k, _ = centers.shape
    bn = 2048
    num_blocks = n // bn

    out = pl.pallas_call(
        functools.partial(_dcn_loss_kernel, n_total=float(n),
                          num_blocks=num_blocks),
        grid=(num_blocks + 1,),
        in_specs=[
            pl.BlockSpec((bn, d),
                         lambda i: (jnp.minimum(i, num_blocks - 1), 0)),
            pl.BlockSpec((k, d), lambda i: (0, 0)),
        ],
        out_specs=pl.BlockSpec((1, 1), lambda i: (0, 0)),
        out_shape=jax.ShapeDtypeStruct((1, 1), jnp.float32),
        scratch_shapes=[
            pltpu.VMEM((d + 8, k), jnp.bfloat16),
            pltpu.VMEM((2, bn, 128), jnp.float32),
            pltpu.SMEM((1, 1), jnp.float32),
        ],
    )(embedded, centers)
    return out[0, 0]
